# baseline (speedup 1.0000x reference)
import functools

import jax
import jax.numpy as jnp
from jax import lax
from jax.experimental import pallas as pl
from jax.experimental.pallas import tpu as pltpu
from jax.experimental.pallas import tpu_sc as plsc

NN = 10000
EE = 320000
HH = 128
GG = 64
ROW = 144
CH = 32
NP = 10240
NTILES = 32
EPT = EE // NTILES
NCHT = EPT // CH
SDL = NCHT * CH + CH + CH
NB = 1000
EB = 4000

_F = jnp.float32


def _tc_pre_body(x_ref, w1t_ref, b1_ref, g1at_ref, x1_ref, u_ref):
    x1 = jnp.tanh(jnp.dot(x_ref[...], w1t_ref[...],
                          preferred_element_type=_F) + b1_ref[...])
    x1_ref[...] = x1
    u_ref[...] = jnp.dot(x1, g1at_ref[...], preferred_element_type=_F)


def _tc_pre(x, w1t, b1, g1at):
    return pl.pallas_call(
        _tc_pre_body,
        grid=(NN // NB,),
        in_specs=[
            pl.BlockSpec((NB, HH), lambda i: (i, 0)),
            pl.BlockSpec((HH, HH), lambda i: (0, 0)),
            pl.BlockSpec((HH,), lambda i: (0,)),
            pl.BlockSpec((HH, HH), lambda i: (0, 0)),
        ],
        out_specs=[
            pl.BlockSpec((NB, HH), lambda i: (i, 0)),
            pl.BlockSpec((NB, HH), lambda i: (i, 0)),
        ],
        out_shape=[
            jax.ShapeDtypeStruct((NN, HH), _F),
            jax.ShapeDtypeStruct((NN, HH), _F),
        ],
    )(x, w1t, b1, g1at)


def _tc_v_body(ea_ref, g1bt_ref, v_ref):
    v_ref[...] = jnp.dot(ea_ref[...], g1bt_ref[...], preferred_element_type=_F)


def _tc_v(ea, g1bt):
    return pl.pallas_call(
        _tc_v_body,
        grid=(EE // EB,),
        in_specs=[
            pl.BlockSpec((EB, 16), lambda i: (i, 0)),
            pl.BlockSpec((16, HH), lambda i: (0, 0)),
        ],
        out_specs=pl.BlockSpec((EB, HH), lambda i: (i, 0)),
        out_shape=jax.ShapeDtypeStruct((EE, HH), _F),
    )(ea, g1bt)


def _lanesum(x):
    lanes = lax.iota(jnp.int32, 16)
    dnums = lax.GatherDimensionNumbers(
        offset_dims=(), collapsed_slice_dims=(0,), start_index_map=(0,))
    for k in (8, 4, 2, 1):
        x = x + lax.gather(x, (lanes ^ k)[:, None], dnums, (1,),
                           mode=lax.GatherScatterMode.PROMISE_IN_BOUNDS)
    return x


def _sc_gate_body(u_h, v_h, sdp_h, attl_h, zer_h, out_h,
                  sdl, src0, src1, dst0, dst1, dsc0, dsc1,
                  ug0, ug1, vb0, vb1, mb0, mb1, attl_v, acc,
                  semA0, semA1, semS0, semS1):
    c = lax.axis_index("c")
    s = lax.axis_index("s")
    wid = s * 2 + c
    rows = NP // 16
    pltpu.sync_copy(zer_h.at[pl.ds(s * rows, rows)],
                    acc.at[pl.ds(s * rows, rows)])
    pltpu.sync_copy(attl_h, attl_v)
    start = NCHT * wid + jnp.minimum(wid, 16)
    n = jnp.where(wid < 16, NCHT + 1, NCHT)
    pltpu.sync_copy(sdp_h.at[pl.ds(start * CH, SDL)], sdl)
    plsc.subcore_barrier()

    lane0 = lax.iota(jnp.int32, 16) == 0
    srcs = (src0, src1)
    dsts = (dst0, dst1)
    dscs = (dsc0, dsc1)
    ugs = (ug0, ug1)
    vbs = (vb0, vb1)
    mbs = (mb0, mb1)
    semAs = (semA0, semA1)
    semSs = (semS0, semS1)

    def unpack(i, p):
        for h in range(CH // 16):
            sd = sdl[pl.ds(i * CH + 16 * h, 16)]
            srcs[p][pl.ds(16 * h, 16)] = sd & 0xFFFF
            dsts[p][pl.ds(16 * h, 16)] = lax.shift_right_logical(sd, 16)

    def fire(i, p):
        pltpu.async_copy(u_h.at[srcs[p]], ugs[p], semAs[p])
        pltpu.async_copy(v_h.at[pl.ds((start + i) * CH, CH)], vbs[p], semAs[p])

    def wait_a(p):
        pltpu.make_async_copy(u_h.at[srcs[p]], ugs[p], semAs[p]).wait()
        pltpu.make_async_copy(v_h.at[pl.ds(0, CH)], vbs[p], semAs[p]).wait()

    def compute(p):
        ug, vb, mb = ugs[p], vbs[p], mbs[p]

        @plsc.parallel_loop(0, CH, 1, unroll=2)
        def edge_body(e):
            ts = []
            acc_p = None
            for j in range(8):
                sl = pl.ds(16 * j, 16)
                z = ug[e, sl] + vb[e, sl]
                ez = jnp.exp(z + z)
                t = (ez - 1.0) / (ez + 1.0)
                ts.append(t)
                q = t * attl_v[sl]
                acc_p = q if acc_p is None else acc_p + q
            wv = jnp.exp(_lanesum(acc_p))
            for j in range(8):
                mb[e, pl.ds(16 * j, 16)] = ts[j] * wv
            mb[e, pl.ds(128, 16)] = jnp.where(lane0, wv, 0.0)

    unpack(0, 0)
    fire(0, 0)

    def chunk_pair(i2, carry):
        for par in (0, 1):
            i = i2 * 2 + par

            @pl.when(i < n)
            def _():
                @pl.when(i + 1 < n)
                def _():
                    unpack(i + 1, 1 - par)
                    fire(i + 1, 1 - par)

                wait_a(par)

                @pl.when(i >= 2)
                def _():
                    pltpu.make_async_copy(
                        mbs[par], acc.at[dscs[par]], semSs[par]).wait()

                compute(par)
                for h in range(CH // 16):
                    dscs[par][pl.ds(16 * h, 16)] = \
                        dsts[par][pl.ds(16 * h, 16)]
                pltpu.async_copy(mbs[par], acc.at[dscs[par]], semSs[par],
                                 add=True)
        return carry

    lax.fori_loop(0, (NCHT + 2) // 2, chunk_pair, 0)
    pltpu.make_async_copy(mb0, acc.at[dsc0], semS0).wait()
    pltpu.make_async_copy(mb1, acc.at[dsc1], semS1).wait()
    plsc.subcore_barrier()
    pltpu.sync_copy(acc.at[pl.ds(s * rows, rows)],
                    out_h.at[c, pl.ds(s * rows, rows)])


def _sc_gate(u, v, sdp, attl, zer):
    mesh = plsc.VectorSubcoreMesh(core_axis_name="c", subcore_axis_name="s",
                                  num_cores=2, num_subcores=16)
    f = pl.kernel(
        _sc_gate_body,
        out_type=jax.ShapeDtypeStruct((2, NP, ROW), _F),
        mesh=mesh,
        compiler_params=pltpu.CompilerParams(use_tc_tiling_on_sc=False),
        scratch_types=[
            pltpu.VMEM((SDL,), jnp.int32),
            pltpu.VMEM((CH,), jnp.int32),
            pltpu.VMEM((CH,), jnp.int32),
            pltpu.VMEM((CH,), jnp.int32),
            pltpu.VMEM((CH,), jnp.int32),
            pltpu.VMEM((CH,), jnp.int32),
            pltpu.VMEM((CH,), jnp.int32),
            pltpu.VMEM((CH, HH), _F),
            pltpu.VMEM((CH, HH), _F),
            pltpu.VMEM((CH, HH), _F),
            pltpu.VMEM((CH, HH), _F),
            pltpu.VMEM((CH, ROW), _F),
            pltpu.VMEM((CH, ROW), _F),
            pltpu.VMEM((HH,), _F),
            pltpu.VMEM_SHARED((NP, ROW), _F),
            pltpu.SemaphoreType.DMA,
            pltpu.SemaphoreType.DMA,
            pltpu.SemaphoreType.DMA,
            pltpu.SemaphoreType.DMA,
        ],
    )
    return f(u, v, sdp, attl, zer)


def _gru(h, x, wiht, whht, bih, bhh):
    gi = jnp.dot(h, wiht, preferred_element_type=_F) + bih
    gh = jnp.dot(x, whht, preferred_element_type=_F) + bhh
    r = jax.nn.sigmoid(gi[:, :HH] + gh[:, :HH])
    z = jax.nn.sigmoid(gi[:, HH:2 * HH] + gh[:, HH:2 * HH])
    n = jnp.tanh(gi[:, 2 * HH:] + r * gh[:, 2 * HH:])
    return (1.0 - z) * n + z * x


def _ln(g, lng, lnb):
    mu = jnp.mean(g, axis=1, keepdims=True)
    d = g - mu
    var = jnp.mean(d * d, axis=1, keepdims=True)
    return d * lax.rsqrt(var + 1e-5) * lng + lnb


def _tc_mid_body(sg_ref, x1_ref, g2t_ref, gb_ref, wih_ref, whh_ref, bih_ref,
                 bhh_ref, lng_ref, lnb_ref, gatwt_ref, gas_ref, gad_ref,
                 x2_ref, hsx_ref, ad2_ref, mxs_ref, mxd_ref):
    ssum = sg_ref[0] + sg_ref[1]
    s = ssum[:, :HH] / (ssum[:, HH:HH + 1] + 1e-16)
    h = jnp.tanh(jnp.dot(s, g2t_ref[...], preferred_element_type=_F)
                 + gb_ref[...])
    x1 = x1_ref[...]
    g = _gru(h, x1, wih_ref[...], whh_ref[...], bih_ref[...], bhh_ref[...])
    x2 = jnp.tanh(_ln(g, lng_ref[...], lnb_ref[...]))
    x2_ref[...] = x2
    hs = jnp.dot(x2, gatwt_ref[...], preferred_element_type=_F)
    a_s = jnp.sum(hs * gas_ref[...][None, :], axis=1)
    a_d = jnp.sum(hs * gad_ref[...][None, :], axis=1)
    hsx_ref[:, :HH] = hs
    lane16 = lax.broadcasted_iota(jnp.int32, (NB, 16), 1)
    hsx_ref[:, HH:] = jnp.where(lane16 == 0, a_s[:, None], 0.0)
    ad2_ref[...] = jnp.where(lane16 == 0, a_d[:, None], 0.0)
    bs = jnp.broadcast_to(jnp.max(a_s), (16,))
    bd = jnp.broadcast_to(jnp.max(a_d), (16,))

    @pl.when(pl.program_id(0) == 0)
    def _():
        mxs_ref[...] = bs
        mxd_ref[...] = bd

    @pl.when(pl.program_id(0) != 0)
    def _():
        mxs_ref[...] = jnp.maximum(mxs_ref[...], bs)
        mxd_ref[...] = jnp.maximum(mxd_ref[...], bd)


def _tc_mid(sg, x1, g2t, gb, wih, whh, bih, bhh, lng, lnb, gatwt, gas, gad):
    full = lambda shape: pl.BlockSpec(shape, lambda i: (0,) * len(shape))
    return pl.pallas_call(
        _tc_mid_body,
        grid=(NN // NB,),
        in_specs=[
            pl.BlockSpec((2, NB, ROW), lambda i: (0, i, 0)),
            pl.BlockSpec((NB, HH), lambda i: (i, 0)),
            full((HH, HH)), full((HH,)),
            full((HH, 3 * HH)), full((HH, 3 * HH)), full((3 * HH,)),
            full((3 * HH,)), full((HH,)), full((HH,)),
            full((HH, HH)), full((HH,)), full((HH,)),
        ],
        out_specs=[
            pl.BlockSpec((NB, HH), lambda i: (i, 0)),
            pl.BlockSpec((NB, ROW), lambda i: (i, 0)),
            pl.BlockSpec((NB, 16), lambda i: (i, 0)),
            pl.BlockSpec((16,), lambda i: (0,)),
            pl.BlockSpec((16,), lambda i: (0,)),
        ],
        out_shape=[
            jax.ShapeDtypeStruct((NN, HH), _F),
            jax.ShapeDtypeStruct((NN, ROW), _F),
            jax.ShapeDtypeStruct((NN, 16), _F),
            jax.ShapeDtypeStruct((16,), _F),
            jax.ShapeDtypeStruct((16,), _F),
        ],
    )(sg, x1, g2t, gb, wih, whh, bih, bhh, lng, lnb, gatwt, gas, gad)


def _sc_gat_body(hs_h, ad_h, sdp_h, mxs_h, mxd_h, zer_h, out_h,
                 sdl, src0, src1, dst0, dst1, dsc0, dsc1,
                 hg0, hg1, dg0, dg1, mb0, mb1, mxs_v, mxd_v, acc,
                 semA0, semA1, semS0, semS1):
    c = lax.axis_index("c")
    s = lax.axis_index("s")
    wid = s * 2 + c
    rows = NP // 16
    pltpu.sync_copy(zer_h.at[pl.ds(s * rows, rows)],
                    acc.at[pl.ds(s * rows, rows)])
    pltpu.sync_copy(mxs_h, mxs_v)
    pltpu.sync_copy(mxd_h, mxd_v)
    start = NCHT * wid + jnp.minimum(wid, 16)
    n = jnp.where(wid < 16, NCHT + 1, NCHT)
    pltpu.sync_copy(sdp_h.at[pl.ds(start * CH, SDL)], sdl)
    plsc.subcore_barrier()

    zmax = mxs_v[...] + mxd_v[...]
    cval = jnp.where(zmax > 0, zmax, 0.01 * zmax)
    lane0 = lax.iota(jnp.int32, 16) == 0
    srcs = (src0, src1)
    dsts = (dst0, dst1)
    dscs = (dsc0, dsc1)
    hgs = (hg0, hg1)
    dgs = (dg0, dg1)
    mbs = (mb0, mb1)
    semAs = (semA0, semA1)
    semSs = (semS0, semS1)

    def unpack(i, p):
        for h in range(CH // 16):
            sd = sdl[pl.ds(i * CH + 16 * h, 16)]
            srcs[p][pl.ds(16 * h, 16)] = sd & 0xFFFF
            dsts[p][pl.ds(16 * h, 16)] = lax.shift_right_logical(sd, 16)

    def fire(i, p):
        pltpu.async_copy(hs_h.at[srcs[p]], hgs[p], semAs[p])
        pltpu.async_copy(ad_h.at[dsts[p]], dgs[p], semAs[p])

    def wait_a(p):
        pltpu.make_async_copy(hs_h.at[srcs[p]], hgs[p], semAs[p]).wait()
        pltpu.make_async_copy(ad_h.at[dsts[p]], dgs[p], semAs[p]).wait()

    def compute(p):
        hg, dg, mb = hgs[p], dgs[p], mbs[p]

        @plsc.parallel_loop(0, CH // 16, 1)
        def grp_body(j2):
            for k in range(16):
                e = j2 * 16 + k
                av = hg[e, pl.ds(HH, 16)]
                dv = dg[e, pl.ds(0, 16)]
                z = av + jnp.broadcast_to(dv[0], (16,))
                zl = jnp.where(z > 0, z, 0.01 * z)
                wf = jnp.exp(zl - cval)
                wv = jnp.broadcast_to(wf[0], (16,))
                for j in range(8):
                    slj = pl.ds(16 * j, 16)
                    mb[e, slj] = hg[e, slj] * wv
                mb[e, pl.ds(128, 16)] = jnp.where(lane0, wv, 0.0)

    unpack(0, 0)
    fire(0, 0)

    def chunk_pair(i2, carry):
        for par in (0, 1):
            i = i2 * 2 + par

            @pl.when(i < n)
            def _():
                @pl.when(i + 1 < n)
                def _():
                    unpack(i + 1, 1 - par)
                    fire(i + 1, 1 - par)

                wait_a(par)

                @pl.when(i >= 2)
                def _():
                    pltpu.make_async_copy(
                        mbs[par], acc.at[dscs[par]], semSs[par]).wait()

                compute(par)
                for h in range(CH // 16):
                    dscs[par][pl.ds(16 * h, 16)] = \
                        dsts[par][pl.ds(16 * h, 16)]
                pltpu.async_copy(mbs[par], acc.at[dscs[par]], semSs[par],
                                 add=True)
        return carry

    lax.fori_loop(0, (NCHT + 2) // 2, chunk_pair, 0)
    pltpu.make_async_copy(mb0, acc.at[dsc0], semS0).wait()
    pltpu.make_async_copy(mb1, acc.at[dsc1], semS1).wait()
    plsc.subcore_barrier()
    pltpu.sync_copy(acc.at[pl.ds(s * rows, rows)],
                    out_h.at[c, pl.ds(s * rows, rows)])


def _sc_gat(hsx, a_d, sdp, mxs, mxd, zer):
    mesh = plsc.VectorSubcoreMesh(core_axis_name="c", subcore_axis_name="s",
                                  num_cores=2, num_subcores=16)
    f = pl.kernel(
        _sc_gat_body,
        out_type=jax.ShapeDtypeStruct((2, NP, ROW), _F),
        mesh=mesh,
        compiler_params=pltpu.CompilerParams(use_tc_tiling_on_sc=False),
        scratch_types=[
            pltpu.VMEM((SDL,), jnp.int32),
            pltpu.VMEM((CH,), jnp.int32),
            pltpu.VMEM((CH,), jnp.int32),
            pltpu.VMEM((CH,), jnp.int32),
            pltpu.VMEM((CH,), jnp.int32),
            pltpu.VMEM((CH,), jnp.int32),
            pltpu.VMEM((CH,), jnp.int32),
            pltpu.VMEM((CH, ROW), _F),
            pltpu.VMEM((CH, ROW), _F),
            pltpu.VMEM((CH, 16), _F),
            pltpu.VMEM((CH, 16), _F),
            pltpu.VMEM((CH, ROW), _F),
            pltpu.VMEM((CH, ROW), _F),
            pltpu.VMEM((16,), _F),
            pltpu.VMEM((16,), _F),
            pltpu.VMEM_SHARED((NP, ROW), _F),
            pltpu.SemaphoreType.DMA,
            pltpu.SemaphoreType.DMA,
            pltpu.SemaphoreType.DMA,
            pltpu.SemaphoreType.DMA,
        ],
    )
    return f(hsx, a_d, sdp, mxs, mxd, zer)


def _tc_fin1_body(sa_ref, x2_ref, gatb_ref, wih_ref, whh_ref, bih_ref,
                  bhh_ref, lng_ref, lnb_ref, molwt_ref,
                  x3_ref, hsm_ref):
    ssum = sa_ref[0] + sa_ref[1]
    s = ssum[:, :HH] / (ssum[:, HH:HH + 1] + 1e-16)
    h = jnp.tanh(s + gatb_ref[...])
    x2 = x2_ref[...]
    g = _gru(h, x2, wih_ref[...], whh_ref[...], bih_ref[...], bhh_ref[...])
    x3 = jnp.tanh(_ln(g, lng_ref[...], lnb_ref[...]))
    x3_ref[...] = x3
    hsm_ref[...] = jnp.dot(x3, molwt_ref[...], preferred_element_type=_F)


def _tc_fin1(sa, x2, gatb, wih, whh, bih, bhh, lng, lnb, molwt):
    full = lambda shape: pl.BlockSpec(shape, lambda i: (0,) * len(shape))
    return pl.pallas_call(
        _tc_fin1_body,
        grid=(NN // NB,),
        in_specs=[
            pl.BlockSpec((2, NB, ROW), lambda i: (0, i, 0)),
            pl.BlockSpec((NB, HH), lambda i: (i, 0)),
            full((HH,)),
            full((HH, 3 * HH)), full((HH, 3 * HH)), full((3 * HH,)),
            full((3 * HH,)), full((HH,)), full((HH,)),
            full((HH, HH)),
        ],
        out_specs=[
            pl.BlockSpec((NB, HH), lambda i: (i, 0)),
            pl.BlockSpec((NB, HH), lambda i: (i, 0)),
        ],
        out_shape=[
            jax.ShapeDtypeStruct((NN, HH), _F),
            jax.ShapeDtypeStruct((NN, HH), _F),
        ],
    )(sa, x2, gatb, wih, whh, bih, bhh, lng, lnb, molwt)


def _tc_fin2_body(x3_ref, hsm_ref, bat_ref, molwt_ref, molas_ref, molad_ref,
                  molb_ref, wih_ref, whh_ref, bih_ref, bhh_ref,
                  l2t_ref, l2b_ref, out_ref):
    bat = bat_ref[...]
    gid = lax.broadcasted_iota(jnp.int32, (GG, NN), 0)
    oh = (gid == bat[None, :]).astype(_F)
    x3 = x3_ref[...]
    out = jnp.tanh(jnp.dot(oh, x3, preferred_element_type=_F))
    hsm = hsm_ref[...]
    asm = jnp.sum(hsm * molas_ref[...][None, :], axis=1)
    for _ in range(2):
        hd = jnp.dot(out, molwt_ref[...], preferred_element_type=_F)
        a_d = jnp.sum(hd * molad_ref[...][None, :], axis=1)
        adn = jnp.sum(oh * a_d[:, None], axis=0)
        z = asm + adn
        zl = jnp.where(z > 0, z, 0.01 * z)
        zmask = jnp.where(oh > 0, zl[None, :], -jnp.inf)
        m = jnp.max(zmask, axis=1)
        m = jnp.where(jnp.isfinite(m), m, 0.0)
        mn = jnp.sum(oh * m[:, None], axis=0)
        ww = jnp.exp(zl - mn)
        den = jnp.sum(oh * ww[None, :], axis=1)
        msg = jnp.dot(oh, ww[:, None] * hsm, preferred_element_type=_F)
        hm = jnp.tanh(msg / (den[:, None] + 1e-16) + molb_ref[...][None, :])
        out = jnp.tanh(_gru(hm, out, wih_ref[...], whh_ref[...],
                            bih_ref[...], bhh_ref[...]))
    out_ref[...] = jnp.dot(out, l2t_ref[...],
                           preferred_element_type=_F) + l2b_ref[...][None, :]


def _tc_fin2(x3, hsm, bat, molwt, molas, molad, molb, wih, whh, bih, bhh,
             l2t, l2b):
    return pl.pallas_call(
        _tc_fin2_body,
        out_shape=jax.ShapeDtypeStruct((GG, HH), _F),
    )(x3, hsm, bat, molwt, molas, molad, molb, wih, whh, bih, bhh, l2t, l2b)


def kernel(x, edge_index, edge_attr, batch, params):
    (lin1_w, lin1_b, g_lin1_w, g_lin2_w, g_att_l, g_att_r, g_bias,
     gru0_wih, gru0_whh, gru0_bih, gru0_bhh, ln0_g, ln0_b,
     gat_w, gat_as, gat_ad, gat_b,
     gru1_wih, gru1_whh, gru1_bih, gru1_bhh, ln1_g, ln1_b,
     mol_w, mol_as, mol_ad, mol_b,
     mgru_wih, mgru_whh, mgru_bih, mgru_bhh, lin2_w, lin2_b) = params

    src = edge_index[0].astype(jnp.int32)
    dst = edge_index[1].astype(jnp.int32)
    bat = batch.astype(jnp.int32)
    zer = jnp.zeros((NP, ROW), _F)
    sdp = jnp.pad((dst << 16) | src, (0, 2 * CH))

    x1, u = _tc_pre(x, lin1_w.T, lin1_b, g_lin1_w[:, :HH].T)
    v = _tc_v(edge_attr, g_lin1_w[:, HH:].T)
    sg = _sc_gate(u, v, sdp, g_att_l, zer)
    x2, hsx, ad2, mxs, mxd = _tc_mid(
        sg, x1, g_lin2_w.T, g_bias, gru0_wih.T, gru0_whh.T, gru0_bih,
        gru0_bhh, ln0_g, ln0_b, gat_w.T, gat_as, gat_ad)
    sa = _sc_gat(hsx, ad2, sdp, mxs, mxd, zer)
    x3, hsm = _tc_fin1(
        sa, x2, gat_b, gru1_wih.T, gru1_whh.T, gru1_bih, gru1_bhh,
        ln1_g, ln1_b, mol_w.T)
    out = _tc_fin2(x3, hsm, bat, mol_w.T, mol_as, mol_ad, mol_b,
                   mgru_wih.T, mgru_whh.T, mgru_bih, mgru_bhh,
                   lin2_w.T, lin2_b)
    return out


# speedup vs baseline: 1.0809x; 1.0809x over previous
import functools

import jax
import jax.numpy as jnp
from jax import lax
from jax.experimental import pallas as pl
from jax.experimental.pallas import tpu as pltpu
from jax.experimental.pallas import tpu_sc as plsc

NN = 10000
EE = 320000
HH = 128
GG = 64
ROW = 144
CH = 32
NP = 10240
NTILES = 32
EPT = EE // NTILES
NCHT = EPT // CH
SDL = NCHT * CH + CH + CH
NB = 1000
EB = 4000

_F = jnp.float32


def _tc_pre_body(x_ref, w1t_ref, b1_ref, g1at_ref, x1_ref, u_ref):
    x1 = jnp.tanh(jnp.dot(x_ref[...], w1t_ref[...],
                          preferred_element_type=_F) + b1_ref[...])
    x1_ref[...] = x1
    u_ref[...] = jnp.dot(x1, g1at_ref[...], preferred_element_type=_F)


def _tc_pre(x, w1t, b1, g1at):
    return pl.pallas_call(
        _tc_pre_body,
        grid=(NN // NB,),
        in_specs=[
            pl.BlockSpec((NB, HH), lambda i: (i, 0)),
            pl.BlockSpec((HH, HH), lambda i: (0, 0)),
            pl.BlockSpec((HH,), lambda i: (0,)),
            pl.BlockSpec((HH, HH), lambda i: (0, 0)),
        ],
        out_specs=[
            pl.BlockSpec((NB, HH), lambda i: (i, 0)),
            pl.BlockSpec((NB, HH), lambda i: (i, 0)),
        ],
        out_shape=[
            jax.ShapeDtypeStruct((NN, HH), _F),
            jax.ShapeDtypeStruct((NN, HH), _F),
        ],
    )(x, w1t, b1, g1at)


def _tc_v_body(ea_ref, g1bt_ref, v_ref):
    v_ref[...] = jnp.dot(ea_ref[...], g1bt_ref[...], preferred_element_type=_F)


def _tc_v(ea, g1bt):
    return pl.pallas_call(
        _tc_v_body,
        grid=(EE // EB,),
        in_specs=[
            pl.BlockSpec((EB, 16), lambda i: (i, 0)),
            pl.BlockSpec((16, HH), lambda i: (0, 0)),
        ],
        out_specs=pl.BlockSpec((EB, HH), lambda i: (i, 0)),
        out_shape=jax.ShapeDtypeStruct((EE, HH), _F),
    )(ea, g1bt)


def _lanesum(x):
    lanes = lax.iota(jnp.int32, 16)
    dnums = lax.GatherDimensionNumbers(
        offset_dims=(), collapsed_slice_dims=(0,), start_index_map=(0,))
    for k in (8, 4, 2, 1):
        x = x + lax.gather(x, (lanes ^ k)[:, None], dnums, (1,),
                           mode=lax.GatherScatterMode.PROMISE_IN_BOUNDS)
    return x


def _sc_gate_body(u_h, v_h, sdp_h, attl_h, zer_h, out_h,
                  sdl, src0, src1, dst0, dst1, dsc0, dsc1,
                  ug0, ug1, vb0, vb1, mb0, mb1, attl_v, acc,
                  semA0, semA1, semS0, semS1):
    c = lax.axis_index("c")
    s = lax.axis_index("s")
    wid = s * 2 + c
    rows = NP // 16
    pltpu.sync_copy(zer_h.at[pl.ds(s * rows, rows)],
                    acc.at[pl.ds(s * rows, rows)])
    pltpu.sync_copy(attl_h, attl_v)
    start = NCHT * wid + jnp.minimum(wid, 16)
    n = jnp.where(wid < 16, NCHT + 1, NCHT)
    pltpu.sync_copy(sdp_h.at[pl.ds(start * CH, SDL)], sdl)
    plsc.subcore_barrier()

    lane0 = lax.iota(jnp.int32, 16) == 0
    srcs = (src0, src1)
    dsts = (dst0, dst1)
    dscs = (dsc0, dsc1)
    ugs = (ug0, ug1)
    vbs = (vb0, vb1)
    mbs = (mb0, mb1)
    semAs = (semA0, semA1)
    semSs = (semS0, semS1)

    def unpack(i, p):
        for h in range(CH // 16):
            sd = sdl[pl.ds(i * CH + 16 * h, 16)]
            srcs[p][pl.ds(16 * h, 16)] = sd & 0xFFFF
            dsts[p][pl.ds(16 * h, 16)] = lax.shift_right_logical(sd, 16)

    def fire(i, p):
        pltpu.async_copy(u_h.at[srcs[p]], ugs[p], semAs[p])
        pltpu.async_copy(v_h.at[pl.ds((start + i) * CH, CH)], vbs[p], semAs[p])

    def wait_a(p):
        pltpu.make_async_copy(u_h.at[srcs[p]], ugs[p], semAs[p]).wait()
        pltpu.make_async_copy(v_h.at[pl.ds(0, CH)], vbs[p], semAs[p]).wait()

    def compute(p):
        ug, vb, mb = ugs[p], vbs[p], mbs[p]

        @plsc.parallel_loop(0, CH, 1, unroll=2)
        def edge_body(e):
            ts = []
            acc_p = None
            for j in range(8):
                sl = pl.ds(16 * j, 16)
                z = ug[e, sl] + vb[e, sl]
                ez = jnp.exp(z + z)
                t = (ez - 1.0) / (ez + 1.0)
                ts.append(t)
                q = t * attl_v[sl]
                acc_p = q if acc_p is None else acc_p + q
            wv = jnp.exp(_lanesum(acc_p))
            for j in range(8):
                mb[e, pl.ds(16 * j, 16)] = ts[j] * wv
            mb[e, pl.ds(128, 16)] = jnp.where(lane0, wv, 0.0)

    unpack(0, 0)
    fire(0, 0)

    def chunk_pair(i2, carry):
        for par in (0, 1):
            i = i2 * 2 + par

            @pl.when(i < n)
            def _():
                @pl.when(i + 1 < n)
                def _():
                    unpack(i + 1, 1 - par)
                    fire(i + 1, 1 - par)

                wait_a(par)

                @pl.when(i >= 2)
                def _():
                    pltpu.make_async_copy(
                        mbs[par], acc.at[dscs[par]], semSs[par]).wait()

                compute(par)
                for h in range(CH // 16):
                    dscs[par][pl.ds(16 * h, 16)] = \
                        dsts[par][pl.ds(16 * h, 16)]
                pltpu.async_copy(mbs[par], acc.at[dscs[par]], semSs[par],
                                 add=True)
        return carry

    lax.fori_loop(0, (NCHT + 2) // 2, chunk_pair, 0)
    pltpu.make_async_copy(mb0, acc.at[dsc0], semS0).wait()
    pltpu.make_async_copy(mb1, acc.at[dsc1], semS1).wait()
    plsc.subcore_barrier()
    pltpu.sync_copy(acc.at[pl.ds(s * rows, rows)],
                    out_h.at[c, pl.ds(s * rows, rows)])


def _sc_gate(u, v, sdp, attl, zer):
    mesh = plsc.VectorSubcoreMesh(core_axis_name="c", subcore_axis_name="s",
                                  num_cores=2, num_subcores=16)
    f = pl.kernel(
        _sc_gate_body,
        out_type=jax.ShapeDtypeStruct((2, NP, ROW), _F),
        mesh=mesh,
        compiler_params=pltpu.CompilerParams(use_tc_tiling_on_sc=False),
        scratch_types=[
            pltpu.VMEM((SDL,), jnp.int32),
            pltpu.VMEM((CH,), jnp.int32),
            pltpu.VMEM((CH,), jnp.int32),
            pltpu.VMEM((CH,), jnp.int32),
            pltpu.VMEM((CH,), jnp.int32),
            pltpu.VMEM((CH,), jnp.int32),
            pltpu.VMEM((CH,), jnp.int32),
            pltpu.VMEM((CH, HH), _F),
            pltpu.VMEM((CH, HH), _F),
            pltpu.VMEM((CH, HH), _F),
            pltpu.VMEM((CH, HH), _F),
            pltpu.VMEM((CH, ROW), _F),
            pltpu.VMEM((CH, ROW), _F),
            pltpu.VMEM((HH,), _F),
            pltpu.VMEM_SHARED((NP, ROW), _F),
            pltpu.SemaphoreType.DMA,
            pltpu.SemaphoreType.DMA,
            pltpu.SemaphoreType.DMA,
            pltpu.SemaphoreType.DMA,
        ],
    )
    return f(u, v, sdp, attl, zer)


def _gru(h, x, wiht, whht, bih, bhh):
    gi = jnp.dot(h, wiht, preferred_element_type=_F) + bih
    gh = jnp.dot(x, whht, preferred_element_type=_F) + bhh
    r = jax.nn.sigmoid(gi[:, :HH] + gh[:, :HH])
    z = jax.nn.sigmoid(gi[:, HH:2 * HH] + gh[:, HH:2 * HH])
    n = jnp.tanh(gi[:, 2 * HH:] + r * gh[:, 2 * HH:])
    return (1.0 - z) * n + z * x


def _ln(g, lng, lnb):
    mu = jnp.mean(g, axis=1, keepdims=True)
    d = g - mu
    var = jnp.mean(d * d, axis=1, keepdims=True)
    return d * lax.rsqrt(var + 1e-5) * lng + lnb


def _tc_mid_body(sg_ref, x1_ref, g2t_ref, gb_ref, wih_ref, whh_ref, bih_ref,
                 bhh_ref, lng_ref, lnb_ref, gatwt_ref, gas_ref,
                 x2_ref, hsx_ref):
    ssum = sg_ref[0] + sg_ref[1]
    s = ssum[:, :HH] / (ssum[:, HH:HH + 1] + 1e-16)
    h = jnp.tanh(jnp.dot(s, g2t_ref[...], preferred_element_type=_F)
                 + gb_ref[...])
    x1 = x1_ref[...]
    g = _gru(h, x1, wih_ref[...], whh_ref[...], bih_ref[...], bhh_ref[...])
    x2 = jnp.tanh(_ln(g, lng_ref[...], lnb_ref[...]))
    x2_ref[...] = x2
    hs = jnp.dot(x2, gatwt_ref[...], preferred_element_type=_F)
    a_s = jnp.sum(hs * gas_ref[...][None, :], axis=1)
    hsx_ref[:, :HH] = hs
    lane16 = lax.broadcasted_iota(jnp.int32, (NB, 16), 1)
    hsx_ref[:, HH:] = jnp.where(lane16 == 0, a_s[:, None], 0.0)


def _tc_mid(sg, x1, g2t, gb, wih, whh, bih, bhh, lng, lnb, gatwt, gas):
    full = lambda shape: pl.BlockSpec(shape, lambda i: (0,) * len(shape))
    return pl.pallas_call(
        _tc_mid_body,
        grid=(NN // NB,),
        in_specs=[
            pl.BlockSpec((2, NB, ROW), lambda i: (0, i, 0)),
            pl.BlockSpec((NB, HH), lambda i: (i, 0)),
            full((HH, HH)), full((HH,)),
            full((HH, 3 * HH)), full((HH, 3 * HH)), full((3 * HH,)),
            full((3 * HH,)), full((HH,)), full((HH,)),
            full((HH, HH)), full((HH,)),
        ],
        out_specs=[
            pl.BlockSpec((NB, HH), lambda i: (i, 0)),
            pl.BlockSpec((NB, ROW), lambda i: (i, 0)),
        ],
        out_shape=[
            jax.ShapeDtypeStruct((NN, HH), _F),
            jax.ShapeDtypeStruct((NN, ROW), _F),
        ],
    )(sg, x1, g2t, gb, wih, whh, bih, bhh, lng, lnb, gatwt, gas)


def _tc_scores_body(hsx_ref, gad_ref, ad_ref, mxs_ref, mxd_ref):
    hs = hsx_ref[:, :HH]
    a_d = jnp.sum(hs * gad_ref[...][None, :], axis=1)
    ad_ref[...] = a_d
    mxs_ref[...] = jnp.broadcast_to(jnp.max(hsx_ref[:, HH]), (16,))
    mxd_ref[...] = jnp.broadcast_to(jnp.max(a_d), (16,))


def _tc_scores(hsx, gad):
    return pl.pallas_call(
        _tc_scores_body,
        out_shape=[
            jax.ShapeDtypeStruct((NN,), _F),
            jax.ShapeDtypeStruct((16,), _F),
            jax.ShapeDtypeStruct((16,), _F),
        ],
    )(hsx, gad)


def _sc_gat_body(hs_h, ad_h, sdp_h, mxs_h, mxd_h, zer_h, out_h,
                 sdl, src0, src1, dst0, dst1, dsc0, dsc1,
                 hg0, hg1, dg0, dg1, mb0, mb1, mxs_v, mxd_v, acc,
                 semA0, semA1, semS0, semS1):
    c = lax.axis_index("c")
    s = lax.axis_index("s")
    wid = s * 2 + c
    rows = NP // 16
    pltpu.sync_copy(zer_h.at[pl.ds(s * rows, rows)],
                    acc.at[pl.ds(s * rows, rows)])
    pltpu.sync_copy(mxs_h, mxs_v)
    pltpu.sync_copy(mxd_h, mxd_v)
    start = NCHT * wid + jnp.minimum(wid, 16)
    n = jnp.where(wid < 16, NCHT + 1, NCHT)
    pltpu.sync_copy(sdp_h.at[pl.ds(start * CH, SDL)], sdl)
    plsc.subcore_barrier()

    zmax = mxs_v[...] + mxd_v[...]
    cval = jnp.where(zmax > 0, zmax, 0.01 * zmax)
    lane0 = lax.iota(jnp.int32, 16) == 0
    srcs = (src0, src1)
    dsts = (dst0, dst1)
    dscs = (dsc0, dsc1)
    hgs = (hg0, hg1)
    dgs = (dg0, dg1)
    mbs = (mb0, mb1)
    semAs = (semA0, semA1)
    semSs = (semS0, semS1)

    def unpack(i, p):
        for h in range(CH // 16):
            sd = sdl[pl.ds(i * CH + 16 * h, 16)]
            srcs[p][pl.ds(16 * h, 16)] = sd & 0xFFFF
            dsts[p][pl.ds(16 * h, 16)] = lax.shift_right_logical(sd, 16)

    def fire(i, p):
        pltpu.async_copy(hs_h.at[srcs[p]], hgs[p], semAs[p])
        pltpu.async_copy(ad_h.at[dsts[p]], dgs[p], semAs[p])

    def wait_a(p):
        pltpu.make_async_copy(hs_h.at[srcs[p]], hgs[p], semAs[p]).wait()
        pltpu.make_async_copy(ad_h.at[dsts[p]], dgs[p], semAs[p]).wait()

    def compute(p):
        hg, dg, mb = hgs[p], dgs[p], mbs[p]

        @plsc.parallel_loop(0, CH // 16, 1)
        def grp_body(j2):
            dv16 = dg[pl.ds(j2 * 16, 16)]
            for k in range(16):
                e = j2 * 16 + k
                av = hg[e, pl.ds(HH, 16)]
                z = av + jnp.broadcast_to(dv16[k], (16,))
                zl = jnp.where(z > 0, z, 0.01 * z)
                wf = jnp.exp(zl - cval)
                wv = jnp.broadcast_to(wf[0], (16,))
                for j in range(8):
                    slj = pl.ds(16 * j, 16)
                    mb[e, slj] = hg[e, slj] * wv
                mb[e, pl.ds(128, 16)] = jnp.where(lane0, wv, 0.0)

    unpack(0, 0)
    fire(0, 0)

    def chunk_pair(i2, carry):
        for par in (0, 1):
            i = i2 * 2 + par

            @pl.when(i < n)
            def _():
                @pl.when(i + 1 < n)
                def _():
                    unpack(i + 1, 1 - par)
                    fire(i + 1, 1 - par)

                wait_a(par)

                @pl.when(i >= 2)
                def _():
                    pltpu.make_async_copy(
                        mbs[par], acc.at[dscs[par]], semSs[par]).wait()

                compute(par)
                for h in range(CH // 16):
                    dscs[par][pl.ds(16 * h, 16)] = \
                        dsts[par][pl.ds(16 * h, 16)]
                pltpu.async_copy(mbs[par], acc.at[dscs[par]], semSs[par],
                                 add=True)
        return carry

    lax.fori_loop(0, (NCHT + 2) // 2, chunk_pair, 0)
    pltpu.make_async_copy(mb0, acc.at[dsc0], semS0).wait()
    pltpu.make_async_copy(mb1, acc.at[dsc1], semS1).wait()
    plsc.subcore_barrier()
    pltpu.sync_copy(acc.at[pl.ds(s * rows, rows)],
                    out_h.at[c, pl.ds(s * rows, rows)])


def _sc_gat(hsx, a_d, sdp, mxs, mxd, zer):
    mesh = plsc.VectorSubcoreMesh(core_axis_name="c", subcore_axis_name="s",
                                  num_cores=2, num_subcores=16)
    f = pl.kernel(
        _sc_gat_body,
        out_type=jax.ShapeDtypeStruct((2, NP, ROW), _F),
        mesh=mesh,
        compiler_params=pltpu.CompilerParams(use_tc_tiling_on_sc=False),
        scratch_types=[
            pltpu.VMEM((SDL,), jnp.int32),
            pltpu.VMEM((CH,), jnp.int32),
            pltpu.VMEM((CH,), jnp.int32),
            pltpu.VMEM((CH,), jnp.int32),
            pltpu.VMEM((CH,), jnp.int32),
            pltpu.VMEM((CH,), jnp.int32),
            pltpu.VMEM((CH,), jnp.int32),
            pltpu.VMEM((CH, ROW), _F),
            pltpu.VMEM((CH, ROW), _F),
            pltpu.VMEM((CH,), _F),
            pltpu.VMEM((CH,), _F),
            pltpu.VMEM((CH, ROW), _F),
            pltpu.VMEM((CH, ROW), _F),
            pltpu.VMEM((16,), _F),
            pltpu.VMEM((16,), _F),
            pltpu.VMEM_SHARED((NP, ROW), _F),
            pltpu.SemaphoreType.DMA,
            pltpu.SemaphoreType.DMA,
            pltpu.SemaphoreType.DMA,
            pltpu.SemaphoreType.DMA,
        ],
    )
    return f(hsx, a_d, sdp, mxs, mxd, zer)


def _tc_fin1_body(sa_ref, x2_ref, gatb_ref, wih_ref, whh_ref, bih_ref,
                  bhh_ref, lng_ref, lnb_ref, molwt_ref,
                  x3_ref, hsm_ref):
    ssum = sa_ref[0] + sa_ref[1]
    s = ssum[:, :HH] / (ssum[:, HH:HH + 1] + 1e-16)
    h = jnp.tanh(s + gatb_ref[...])
    x2 = x2_ref[...]
    g = _gru(h, x2, wih_ref[...], whh_ref[...], bih_ref[...], bhh_ref[...])
    x3 = jnp.tanh(_ln(g, lng_ref[...], lnb_ref[...]))
    x3_ref[...] = x3
    hsm_ref[...] = jnp.dot(x3, molwt_ref[...], preferred_element_type=_F)


def _tc_fin1(sa, x2, gatb, wih, whh, bih, bhh, lng, lnb, molwt):
    full = lambda shape: pl.BlockSpec(shape, lambda i: (0,) * len(shape))
    return pl.pallas_call(
        _tc_fin1_body,
        grid=(NN // NB,),
        in_specs=[
            pl.BlockSpec((2, NB, ROW), lambda i: (0, i, 0)),
            pl.BlockSpec((NB, HH), lambda i: (i, 0)),
            full((HH,)),
            full((HH, 3 * HH)), full((HH, 3 * HH)), full((3 * HH,)),
            full((3 * HH,)), full((HH,)), full((HH,)),
            full((HH, HH)),
        ],
        out_specs=[
            pl.BlockSpec((NB, HH), lambda i: (i, 0)),
            pl.BlockSpec((NB, HH), lambda i: (i, 0)),
        ],
        out_shape=[
            jax.ShapeDtypeStruct((NN, HH), _F),
            jax.ShapeDtypeStruct((NN, HH), _F),
        ],
    )(sa, x2, gatb, wih, whh, bih, bhh, lng, lnb, molwt)


def _tc_fin2_body(x3_ref, hsm_ref, bat_ref, molwt_ref, molas_ref, molad_ref,
                  molb_ref, wih_ref, whh_ref, bih_ref, bhh_ref,
                  l2t_ref, l2b_ref, out_ref):
    bat = bat_ref[...]
    gid = lax.broadcasted_iota(jnp.int32, (GG, NN), 0)
    oh = (gid == bat[None, :]).astype(_F)
    x3 = x3_ref[...]
    out = jnp.tanh(jnp.dot(oh, x3, preferred_element_type=_F))
    hsm = hsm_ref[...]
    asm = jnp.sum(hsm * molas_ref[...][None, :], axis=1)
    for _ in range(2):
        hd = jnp.dot(out, molwt_ref[...], preferred_element_type=_F)
        a_d = jnp.sum(hd * molad_ref[...][None, :], axis=1)
        adn = jnp.sum(oh * a_d[:, None], axis=0)
        z = asm + adn
        zl = jnp.where(z > 0, z, 0.01 * z)
        zmask = jnp.where(oh > 0, zl[None, :], -jnp.inf)
        m = jnp.max(zmask, axis=1)
        m = jnp.where(jnp.isfinite(m), m, 0.0)
        mn = jnp.sum(oh * m[:, None], axis=0)
        ww = jnp.exp(zl - mn)
        den = jnp.sum(oh * ww[None, :], axis=1)
        msg = jnp.dot(oh, ww[:, None] * hsm, preferred_element_type=_F)
        hm = jnp.tanh(msg / (den[:, None] + 1e-16) + molb_ref[...][None, :])
        out = jnp.tanh(_gru(hm, out, wih_ref[...], whh_ref[...],
                            bih_ref[...], bhh_ref[...]))
    out_ref[...] = jnp.dot(out, l2t_ref[...],
                           preferred_element_type=_F) + l2b_ref[...][None, :]


def _tc_fin2(x3, hsm, bat, molwt, molas, molad, molb, wih, whh, bih, bhh,
             l2t, l2b):
    return pl.pallas_call(
        _tc_fin2_body,
        out_shape=jax.ShapeDtypeStruct((GG, HH), _F),
    )(x3, hsm, bat, molwt, molas, molad, molb, wih, whh, bih, bhh, l2t, l2b)


def kernel(x, edge_index, edge_attr, batch, params):
    (lin1_w, lin1_b, g_lin1_w, g_lin2_w, g_att_l, g_att_r, g_bias,
     gru0_wih, gru0_whh, gru0_bih, gru0_bhh, ln0_g, ln0_b,
     gat_w, gat_as, gat_ad, gat_b,
     gru1_wih, gru1_whh, gru1_bih, gru1_bhh, ln1_g, ln1_b,
     mol_w, mol_as, mol_ad, mol_b,
     mgru_wih, mgru_whh, mgru_bih, mgru_bhh, lin2_w, lin2_b) = params

    src = edge_index[0].astype(jnp.int32)
    dst = edge_index[1].astype(jnp.int32)
    bat = batch.astype(jnp.int32)
    zer = jnp.zeros((NP, ROW), _F)
    sdp = jnp.pad((dst << 16) | src, (0, 2 * CH))

    x1, u = _tc_pre(x, lin1_w.T, lin1_b, g_lin1_w[:, :HH].T)
    v = _tc_v(edge_attr, g_lin1_w[:, HH:].T)
    sg = _sc_gate(u, v, sdp, g_att_l, zer)
    x2, hsx = _tc_mid(
        sg, x1, g_lin2_w.T, g_bias, gru0_wih.T, gru0_whh.T, gru0_bih,
        gru0_bhh, ln0_g, ln0_b, gat_w.T, gat_as)
    a_d, mxs, mxd = _tc_scores(hsx, gat_ad)
    sa = _sc_gat(hsx, a_d, sdp, mxs, mxd, zer)
    x3, hsm = _tc_fin1(
        sa, x2, gat_b, gru1_wih.T, gru1_whh.T, gru1_bih, gru1_bhh,
        ln1_g, ln1_b, mol_w.T)
    out = _tc_fin2(x3, hsm, bat, mol_w.T, mol_as, mol_ad, mol_b,
                   mgru_wih.T, mgru_whh.T, mgru_bih, mgru_bhh,
                   lin2_w.T, lin2_b)
    return out
